# Initial kernel scaffold; baseline (speedup 1.0000x reference)
import jax
import jax.numpy as jnp
from jax.experimental import pallas as pl


def kernel(x, edge_index, W1, a_src1, a_dst1, b1, Ws1, bs1, W2, a_src2, a_dst2, b2, Ws2, bs2, W3, a_src3, a_dst3, b3, Ws3, bs3):
    raise NotImplementedError("write your pallas kernel here")


# speedup vs baseline: 4.1463x; 4.1463x over previous
import functools

import jax
import jax.numpy as jnp
from jax.experimental import pallas as pl


def _mm_kernel(x_ref, w_ref, o_ref):
    o_ref[...] = jax.lax.dot_general(
        x_ref[...], w_ref[...], (((1,), (0,)), ((), ())),
        preferred_element_type=jnp.float32,
        precision=jax.lax.Precision.DEFAULT)


@functools.partial(jax.jit, static_argnames=("bm", "bn"))
def _mm(x, w, bm=1000, bn=256):
    m, k = x.shape
    k2, n = w.shape
    grid = (m // bm, n // bn)
    return pl.pallas_call(
        _mm_kernel,
        grid=grid,
        in_specs=[pl.BlockSpec((bm, k), lambda i, j: (i, 0)),
                  pl.BlockSpec((k, bn), lambda i, j: (0, j))],
        out_specs=pl.BlockSpec((bm, bn), lambda i, j: (i, j)),
        out_shape=jax.ShapeDtypeStruct((m, n), jnp.float32),
    )(x, w)


def _gat_layer(x, src, dst, W, a_src, a_dst, b, Ws, bs, heads, ch, concat,
               elu):
    n = x.shape[0]
    hw = _mm(x, jnp.concatenate([W, Ws], axis=1))
    h = hw[:, :heads * ch]
    skip = hw[:, heads * ch:]
    h3 = h.reshape(n, heads, ch)
    asrc = (h3 * a_src[None]).sum(-1)
    adst = (h3 * a_dst[None]).sum(-1)
    Amax = asrc.max(0)
    s = jax.nn.leaky_relu(adst + Amax[None], 0.2)
    wself = jnp.exp(jax.nn.leaky_relu(asrc + adst, 0.2) - s)
    e = jax.nn.leaky_relu(asrc[src] + adst[dst], 0.2)
    w = jnp.exp(e - s[dst])
    denom = jax.ops.segment_sum(w, dst, num_segments=n) + wself
    msg = (h3[src] * w[:, :, None]).reshape(-1, heads * ch)
    out = jax.ops.segment_sum(msg, dst, num_segments=n).reshape(n, heads, ch)
    out = (out + h3 * wself[:, :, None]) / (denom[:, :, None] + 1e-30)
    if concat:
        out = out.reshape(n, heads * ch) + b
    else:
        out = out.mean(1) + b
    out = out + skip + bs
    return jax.nn.elu(out) if elu else out


def kernel(x, edge_index, W1, a_src1, a_dst1, b1, Ws1, bs1,
           W2, a_src2, a_dst2, b2, Ws2, bs2,
           W3, a_src3, a_dst3, b3, Ws3, bs3):
    src, dst = edge_index[0], edge_index[1]
    h = _gat_layer(x, src, dst, W1, a_src1, a_dst1, b1, Ws1, bs1,
                   4, 256, True, True)
    h = _gat_layer(h, src, dst, W2, a_src2, a_dst2, b2, Ws2, bs2,
                   4, 256, True, True)
    return _gat_layer(h, src, dst, W3, a_src3, a_dst3, b3, Ws3, bs3,
                      6, 256, False, False)
